# baseline (speedup 1.0000x reference)
import functools

import jax
import jax.numpy as jnp
from jax import lax
from jax.experimental import pallas as pl
from jax.experimental.pallas import tpu as pltpu
from jax.experimental.pallas import tpu_sc as plsc

PIECES = ((0, 128), (128, 72))


@functools.cache
def _build(n_batch, n_hist, n_table_rows, dim):
    info = plsc.get_sparse_core_info()
    nc, ns = info.num_cores, info.num_subcores
    nw = nc * ns
    rows_w = n_batch // nw
    assert rows_w * nw == n_batch and rows_w % 8 == 0
    assert sum(sz for _, sz in PIECES) == n_hist
    assert all(off % 128 == 0 and sz <= 128 for off, sz in PIECES)

    base = (n_table_rows // (8 * ns)) * 8
    k8 = (n_table_rows - base * ns) // 8
    sizes = [base + (8 if k < k8 else 0) for k in range(ns)]
    starts = [sum(sizes[:k]) for k in range(ns)]
    assert sum(sizes) == n_table_rows and all(s % 8 == 0 for s in starts + sizes)

    mesh = plsc.VectorSubcoreMesh(core_axis_name="c", subcore_axis_name="s")

    @functools.partial(
        pl.kernel,
        mesh=mesh,
        out_type=jax.ShapeDtypeStruct((n_batch, n_hist, dim), jnp.float32),
        scratch_types=[
            pltpu.VMEM((rows_w, n_hist), jnp.int32),
            pltpu.VMEM((PIECES[0][1], dim), jnp.float32),
            pltpu.VMEM((PIECES[1][1], dim), jnp.float32),
            pltpu.VMEM_SHARED((n_table_rows, dim), jnp.float32),
            pltpu.SemaphoreType.DMA,
        ]
        + [pltpu.SemaphoreType.DMA] * 4,
    )
    def gather_kernel(idx_hbm, table_hbm, out_hbm, idx_v, buf0, buf1, table_sh,
                      psem, *sems):
        bufs = (buf0, buf1)
        gsem, wsem = sems[:2], sems[2:]
        sid = lax.axis_index("s")
        wid = sid * nc + lax.axis_index("c")
        row0 = pl.multiple_of(wid * rows_w, 8)

        for k in range(ns):

            @pl.when(sid == k)
            def _(k=k):
                pltpu.async_copy(
                    table_hbm.at[pl.ds(starts[k], sizes[k])],
                    table_sh.at[pl.ds(starts[k], sizes[k])],
                    psem,
                )

        pltpu.sync_copy(idx_hbm.at[pl.ds(row0, rows_w)], idx_v)

        def gfire(table, j, b):
            off, sz = PIECES[b]
            pltpu.async_copy(
                table.at[idx_v.at[j, pl.ds(off, sz)]], bufs[b], gsem[b]
            )

        def gwait(b):
            pltpu.make_async_copy(
                table_hbm.at[idx_v.at[0, pl.ds(PIECES[b][0], PIECES[b][1])]],
                bufs[b],
                gsem[b],
            ).wait()

        def wfire(j, b):
            off, sz = PIECES[b]
            pltpu.async_copy(bufs[b], out_hbm.at[row0 + j, pl.ds(off, sz)], wsem[b])

        def wwait(b):
            off, sz = PIECES[b]
            pltpu.make_async_copy(
                bufs[b], out_hbm.at[0, pl.ds(off, sz)], wsem[b]
            ).wait()

        for b in range(2):
            gfire(table_hbm, 0, b)

        for k in range(ns):

            @pl.when(sid == k)
            def _(k=k):
                pltpu.make_async_copy(
                    table_hbm.at[pl.ds(starts[k], sizes[k])],
                    table_sh.at[pl.ds(starts[k], sizes[k])],
                    psem,
                ).wait()

        plsc.subcore_barrier()

        def body(j, carry):
            for b in range(2):
                gwait(b)
                wfire(j, b)

                @pl.when(j + 1 < rows_w)
                def _():
                    wwait(b)
                    gfire(table_sh, j + 1, b)

            return carry

        lax.fori_loop(0, rows_w, body, 0, unroll=False)
        for b in range(2):
            wwait(b)

    return gather_kernel, nw


def kernel(x, embedding):
    n_batch, n_hist = x.shape
    n_table_rows, dim = embedding.shape
    gather_kernel, _ = _build(n_batch, n_hist, n_table_rows, dim)
    return gather_kernel(x, embedding)


# speedup vs baseline: 1.1715x; 1.1715x over previous
import functools

import jax
import jax.numpy as jnp
from jax import lax
from jax.experimental import pallas as pl
from jax.experimental.pallas import tpu as pltpu
from jax.experimental.pallas import tpu_sc as plsc

DIM = 128
CHUNK = 128
NBUF = 2
LAG = 2
HBM_CHUNKS = 2


@functools.cache
def _build(n_rows, n_table_rows, dim):
    info = plsc.get_sparse_core_info()
    nc, ns = info.num_cores, info.num_subcores
    nw = nc * ns
    n_chunks = n_rows // (nw * CHUNK)
    assert n_chunks * nw * CHUNK == n_rows
    assert HBM_CHUNKS % NBUF == 0 and LAG <= NBUF <= HBM_CHUNKS
    n_rest = n_chunks - HBM_CHUNKS
    n_main = (n_rest // NBUF) * NBUF

    pre_rows = -(-n_table_rows // (8 * ns)) * 8
    pre_last = n_table_rows - pre_rows
    assert pre_rows % 8 == 0 and pre_last % 8 == 0 and pre_last >= 0

    mesh = plsc.VectorSubcoreMesh(core_axis_name="c", subcore_axis_name="s")

    @functools.partial(
        pl.kernel,
        mesh=mesh,
        out_type=jax.ShapeDtypeStruct((nw, n_chunks, CHUNK, dim), jnp.float32),
        scratch_types=[
            pltpu.VMEM((n_chunks, CHUNK), jnp.int32),
            pltpu.VMEM((NBUF, CHUNK, dim), jnp.float32),
            pltpu.VMEM_SHARED((n_table_rows, dim), jnp.float32),
            pltpu.SemaphoreType.DMA,
        ]
        + [pltpu.SemaphoreType.DMA] * (2 * NBUF),
    )
    def gather_kernel(idx_hbm, table_hbm, out_hbm, idx_v, rows_v, table_sh, psem, *sems):
        gsem, wsem = sems[:NBUF], sems[NBUF:]
        sid = lax.axis_index("s")
        wid = sid * nc + lax.axis_index("c")

        pre_start = pl.multiple_of(jnp.minimum(sid * pre_rows, pre_last), 8)
        pltpu.async_copy(
            table_hbm.at[pl.ds(pre_start, pre_rows)],
            table_sh.at[pl.ds(pre_start, pre_rows)],
            psem,
        )

        pltpu.sync_copy(idx_hbm.at[wid], idx_v)

        def gfire_hbm(c, b):
            pltpu.async_copy(table_hbm.at[idx_v.at[c]], rows_v.at[b], gsem[b])

        def gfire_sp(c, b):
            pltpu.async_copy(table_sh.at[idx_v.at[c]], rows_v.at[b], gsem[b])

        def gwait(b):
            pltpu.make_async_copy(
                table_hbm.at[idx_v.at[0]], rows_v.at[b], gsem[b]
            ).wait()

        def wfire(c, b):
            pltpu.async_copy(rows_v.at[b], out_hbm.at[wid, c], wsem[b])

        def wwait(b):
            pltpu.make_async_copy(rows_v.at[b], out_hbm.at[wid, 0], wsem[b]).wait()

        for b in range(LAG):
            gfire_hbm(b, b)

        def static_step(c, gfire_fn):
            b = c % NBUF
            gwait(b)
            wfire(c, b)
            if c + LAG < n_chunks:
                bn = (b + LAG) % NBUF
                if c + LAG >= NBUF:
                    wwait(bn)
                gfire_fn(c + LAG, bn)

        for c in range(HBM_CHUNKS - LAG):
            static_step(c, gfire_hbm)

        pltpu.make_async_copy(
            table_hbm.at[pl.ds(0, pre_rows)],
            table_sh.at[pl.ds(0, pre_rows)],
            psem,
        ).wait()
        plsc.subcore_barrier()

        for c in range(HBM_CHUNKS - LAG, HBM_CHUNKS):
            static_step(c, gfire_sp)

        def body(j, carry):
            for b in range(NBUF):
                c = HBM_CHUNKS + j * NBUF + b
                gwait(b)
                wfire(c, b)
                bn = (b + LAG) % NBUF

                @pl.when(c + LAG < n_chunks)
                def _():
                    wwait(bn)
                    gfire_sp(c + LAG, bn)

            return carry

        lax.fori_loop(0, n_main // NBUF, body, 0, unroll=False)
        for c in range(HBM_CHUNKS + n_main, n_chunks):
            static_step(c, gfire_sp)
        for b in range(NBUF):
            wwait(b)

    return gather_kernel, nw, n_chunks


def kernel(x, embedding):
    b, h = x.shape
    n_table_rows, dim = embedding.shape
    n_rows = b * h
    gather_kernel, nw, n_chunks = _build(n_rows, n_table_rows, dim)
    idx = x.reshape(nw, n_chunks, CHUNK)
    out = gather_kernel(idx, embedding)
    return out.reshape(b, h, dim)
